# Initial kernel scaffold; baseline (speedup 1.0000x reference)
import jax
import jax.numpy as jnp
from jax.experimental import pallas as pl


def kernel(x, edge_index, W1, b1, W2, b2):
    raise NotImplementedError("write your pallas kernel here")


# speedup vs baseline: 33.5675x; 33.5675x over previous
import functools

import jax
import jax.numpy as jnp
from jax import lax
from jax.experimental import pallas as pl
from jax.experimental.pallas import tpu as pltpu
from jax.experimental.pallas import tpu_sc as plsc

NC = 2
NS = 16
NW = NC * NS
CHUNK = 128
GROUP = 8
F = 16


def _edge_pass(n_pad, e_chunks):
    cpw = e_chunks // NW
    ngroups = cpw // GROUP
    rpt = n_pad // NS
    mesh = plsc.VectorSubcoreMesh(core_axis_name="c", subcore_axis_name="s")

    @functools.partial(
        pl.kernel,
        out_type=jax.ShapeDtypeStruct((NC, n_pad, F), jnp.float32),
        mesh=mesh,
        scratch_types=[
            pltpu.VMEM((GROUP, CHUNK), jnp.int32),
            pltpu.VMEM((GROUP, CHUNK), jnp.int32),
            pltpu.VMEM((GROUP * CHUNK, F), jnp.float32),
            pltpu.VMEM_SHARED((n_pad, F), jnp.float32),
            pltpu.SemaphoreType.DMA,
        ],
        compiler_params=pltpu.CompilerParams(use_tc_tiling_on_sc=False),
    )
    def ek(src_hbm, dst_hbm, hs_hbm, zeros_hbm, out_hbm, src_v, dst_v, rows_v,
           accum_sh, sem):
        c = lax.axis_index("c")
        s = lax.axis_index("s")
        wid = s * NC + c
        pltpu.sync_copy(zeros_hbm.at[pl.ds(s * rpt, rpt)],
                        accum_sh.at[pl.ds(s * rpt, rpt)])
        plsc.subcore_barrier()

        def group_body(g, carry):
            row0 = wid * cpw + g * GROUP
            pltpu.sync_copy(src_hbm.at[pl.ds(row0, GROUP)], src_v)
            pltpu.sync_copy(dst_hbm.at[pl.ds(row0, GROUP)], dst_v)
            descs = [
                pltpu.async_copy(hs_hbm.at[src_v.at[j]],
                                 rows_v.at[pl.ds(j * CHUNK, CHUNK)], sem)
                for j in range(GROUP)
            ]
            for d in descs:
                d.wait()
            for j in range(GROUP):
                pltpu.sync_copy(rows_v.at[pl.ds(j * CHUNK, CHUNK)],
                                accum_sh.at[dst_v.at[j]], add=True)
            return carry

        lax.fori_loop(0, ngroups, group_body, 0)
        plsc.subcore_barrier()
        pltpu.sync_copy(accum_sh.at[pl.ds(s * rpt, rpt)],
                        out_hbm.at[c].at[pl.ds(s * rpt, rpt)])

    return ek


def _deg_pass(n_pad, e_chunks):
    cpw = e_chunks // NW
    ngroups = cpw // GROUP
    rpt = n_pad // NS
    mesh = plsc.VectorSubcoreMesh(core_axis_name="c", subcore_axis_name="s")

    @functools.partial(
        pl.kernel,
        out_type=jax.ShapeDtypeStruct((NC, n_pad, F), jnp.float32),
        mesh=mesh,
        scratch_types=[
            pltpu.VMEM((GROUP, CHUNK), jnp.int32),
            pltpu.VMEM((CHUNK, F), jnp.float32),
            pltpu.VMEM_SHARED((n_pad, F), jnp.float32),
        ],
        compiler_params=pltpu.CompilerParams(use_tc_tiling_on_sc=False),
    )
    def dk(dst_hbm, ones_hbm, zeros_hbm, out_hbm, dst_v, ones_v, accum_sh):
        c = lax.axis_index("c")
        s = lax.axis_index("s")
        wid = s * NC + c
        pltpu.sync_copy(ones_hbm, ones_v)
        pltpu.sync_copy(zeros_hbm.at[pl.ds(s * rpt, rpt)],
                        accum_sh.at[pl.ds(s * rpt, rpt)])
        plsc.subcore_barrier()

        def group_body(g, carry):
            row0 = wid * cpw + g * GROUP
            pltpu.sync_copy(dst_hbm.at[pl.ds(row0, GROUP)], dst_v)
            for j in range(GROUP):
                pltpu.sync_copy(ones_v, accum_sh.at[dst_v.at[j]], add=True)
            return carry

        lax.fori_loop(0, ngroups, group_body, 0)
        plsc.subcore_barrier()
        pltpu.sync_copy(accum_sh.at[pl.ds(s * rpt, rpt)],
                        out_hbm.at[c].at[pl.ds(s * rpt, rpt)])

    return dk


def _tc_stage1(n_pad, d):
    def body(degp_ref, x_ref, w1_ref, hs_ref, dis_ref):
        deg = degp_ref[0] + degp_ref[1] + 1.0
        dis = lax.rsqrt(deg)
        h = jnp.dot(x_ref[...], w1_ref[...],
                    preferred_element_type=jnp.float32,
                    precision=lax.Precision.HIGHEST)
        hs_ref[...] = h * dis
        dis_ref[...] = dis

    return pl.pallas_call(
        body,
        out_shape=[
            jax.ShapeDtypeStruct((n_pad, F), jnp.float32),
            jax.ShapeDtypeStruct((n_pad, F), jnp.float32),
        ],
    )


def _tc_stage2(n_pad):
    def body(accp_ref, hs1_ref, dis_ref, w2_ref, b1_ref, hs2_ref):
        a = accp_ref[0] + accp_ref[1] + hs1_ref[...]
        out1 = dis_ref[...] * a + b1_ref[...]
        r = jnp.maximum(out1, 0.0)
        h2 = jnp.dot(r, w2_ref[...],
                     preferred_element_type=jnp.float32,
                     precision=lax.Precision.HIGHEST)
        hs2_ref[...] = h2 * dis_ref[...]

    return pl.pallas_call(
        body,
        out_shape=jax.ShapeDtypeStruct((n_pad, F), jnp.float32),
    )


def _tc_stage3(n_pad, c_out):
    def body(accp_ref, hs2_ref, dis_ref, b2_ref, out_ref):
        a = accp_ref[0] + accp_ref[1] + hs2_ref[...]
        v = dis_ref[...] * a + b2_ref[...]
        col = lax.broadcasted_iota(jnp.int32, (n_pad, F), 1)
        masked = jnp.where(col < c_out, v, -1e30)
        m = jnp.max(masked, axis=1, keepdims=True)
        e = jnp.exp(masked - m)
        ssum = jnp.sum(e, axis=1, keepdims=True)
        out_ref[...] = v - m - jnp.log(ssum)

    return pl.pallas_call(
        body,
        out_shape=jax.ShapeDtypeStruct((n_pad, F), jnp.float32),
    )


def kernel(x, edge_index, W1, b1, W2, b2):
    n, d = x.shape
    h_dim = W1.shape[1]
    c_out = W2.shape[1]
    e = edge_index.shape[1]
    assert h_dim == F and c_out <= F

    n_pad = ((n + 1 + NS * 8 - 1) // (NS * 8)) * (NS * 8)
    step = NW * CHUNK * GROUP
    e_pad = ((e + step - 1) // step) * step
    e_chunks = e_pad // CHUNK

    src = edge_index[0]
    dst = edge_index[1]
    dummy = jnp.full((e_pad - e,), n, dtype=jnp.int32)
    src2d = jnp.concatenate([src, dummy]).reshape(e_chunks, CHUNK)
    dst2d = jnp.concatenate([dst, dummy]).reshape(e_chunks, CHUNK)

    x_pad = jnp.zeros((n_pad, d), jnp.float32).at[:n].set(x)
    zeros16 = jnp.zeros((n_pad, F), jnp.float32)
    ones_chunk = jnp.ones((CHUNK, F), jnp.float32)
    w2p = jnp.zeros((h_dim, F), jnp.float32).at[:, :c_out].set(W2)
    b1r = b1.reshape(1, F)
    b2r = jnp.zeros((1, F), jnp.float32).at[0, :c_out].set(b2)

    degp = _deg_pass(n_pad, e_chunks)(dst2d, ones_chunk, zeros16)
    hs1, dis = _tc_stage1(n_pad, d)(degp, x_pad, W1)
    accp1 = _edge_pass(n_pad, e_chunks)(src2d, dst2d, hs1, zeros16)
    hs2 = _tc_stage2(n_pad)(accp1, hs1, dis, w2p, b1r)
    accp2 = _edge_pass(n_pad, e_chunks)(src2d, dst2d, hs2, zeros16)
    full = _tc_stage3(n_pad, c_out)(accp2, hs2, dis, b2r)
    return full[:n, :c_out]
